# Initial kernel scaffold; baseline (speedup 1.0000x reference)
import jax
import jax.numpy as jnp
from jax.experimental import pallas as pl


def kernel(x):
    raise NotImplementedError("write your pallas kernel here")


# speedup vs baseline: 6.4672x; 6.4672x over previous
import jax
import jax.numpy as jnp
from jax.experimental import pallas as pl

_B, _GH, _GW, _D = 64, 24, 24, 768
_N = _GH * _GW
_LO, _HI = _GW, _N - _GW
_M = _HI - _LO
_THRESH_SQ = 1.0
_S = 0.5


def _interior_col():
    p = jax.lax.broadcasted_iota(jnp.int32, (_M, 1), 0) + _LO
    i = p // _GW
    j = p % _GW
    ok = (i >= 1) & (i <= _GH - 2) & (j >= 1) & (j <= _GW - 2)
    return ok


def _mask_kernel(x_ref, m_ref):
    b = pl.program_id(0)
    xb = x_ref[0]
    d1 = xb[1:, :] - xb[:-1, :]
    s1 = jnp.sum(d1 * d1, axis=1, keepdims=True)
    d24 = xb[_GW:, :] - xb[:-_GW, :]
    s24 = jnp.sum(d24 * d24, axis=1, keepdims=True)
    e1 = (s1 > _THRESH_SQ).astype(jnp.float32)
    e24 = (s24 > _THRESH_SQ).astype(jnp.float32)
    cell = jnp.maximum(
        jnp.maximum(e1[_LO - 1:_HI - 1, :], e1[_LO:_HI, :]),
        jnp.maximum(e24[0:_M, :], e24[_LO:_HI, :]),
    )

    @pl.when(b == 0)
    def _init():
        m_ref[...] = cell

    @pl.when(b != 0)
    def _acc():
        m_ref[...] = jnp.maximum(m_ref[...], cell)


def _blend_kernel(m_ref, x_ref, o_ref):
    xb = x_ref[0]
    center = xb[_LO:_HI, :]
    avg = (xb[_LO - _GW:_HI - _GW, :] + xb[_LO + _GW:_HI + _GW, :]
           + xb[_LO - 1:_HI - 1, :] + xb[_LO + 1:_HI + 1, :]) * 0.25
    blended = (1.0 - _S) * center + _S * avg
    cond = (m_ref[...] > 0.5) & _interior_col()
    o_ref[0] = xb
    o_ref[0, _LO:_HI, :] = jnp.where(cond, blended, center)


def kernel(x):
    mask = pl.pallas_call(
        _mask_kernel,
        grid=(_B,),
        in_specs=[pl.BlockSpec((1, _N, _D), lambda b: (b, 0, 0))],
        out_specs=pl.BlockSpec((_M, 1), lambda b: (0, 0)),
        out_shape=jax.ShapeDtypeStruct((_M, 1), jnp.float32),
    )(x)
    out = pl.pallas_call(
        _blend_kernel,
        grid=(_B,),
        in_specs=[pl.BlockSpec((_M, 1), lambda b: (0, 0)),
                  pl.BlockSpec((1, _N, _D), lambda b: (b, 0, 0))],
        out_specs=pl.BlockSpec((1, _N, _D), lambda b: (b, 0, 0)),
        out_shape=jax.ShapeDtypeStruct((_B, _N, _D), jnp.float32),
    )(mask, x)
    return out


# speedup vs baseline: 8.6398x; 1.3360x over previous
import jax
import jax.numpy as jnp
from jax.experimental import pallas as pl
from jax.experimental.pallas import tpu as pltpu

_B, _GH, _GW, _D = 64, 24, 24, 768
_N = _GH * _GW
_LO, _HI = _GW, _N - _GW
_M = _HI - _LO
_NI = (_GH - 2) * (_GW - 2)
_THRESH_SQ = 1.0
_S = 0.5


def _interior_col():
    p = jax.lax.broadcasted_iota(jnp.int32, (_M, 1), 0) + _LO
    i = p // _GW
    j = p % _GW
    return (i >= 1) & (i <= _GH - 2) & (j >= 1) & (j <= _GW - 2)


def _fused_kernel(x_ref, o_ref, m_ref, am_ref):
    b = pl.program_id(0)
    xb = x_ref[0]
    d1 = xb[1:, :] - xb[:-1, :]
    s1 = jnp.sum(d1 * d1, axis=1, keepdims=True)
    d24 = xb[_GW:, :] - xb[:-_GW, :]
    s24 = jnp.sum(d24 * d24, axis=1, keepdims=True)
    e1 = (s1 > _THRESH_SQ).astype(jnp.float32)
    e24 = (s24 > _THRESH_SQ).astype(jnp.float32)
    cell = jnp.maximum(
        jnp.maximum(e1[_LO - 1:_HI - 1, :], e1[_LO:_HI, :]),
        jnp.maximum(e24[0:_M, :], e24[_LO:_HI, :]),
    )

    interior = _interior_col()
    center = xb[_LO:_HI, :]
    avg = (xb[_LO - _GW:_HI - _GW, :] + xb[_LO + _GW:_HI + _GW, :]
           + xb[_LO - 1:_HI - 1, :] + xb[_LO + 1:_HI + 1, :]) * 0.25
    blended = (1.0 - _S) * center + _S * avg
    o_ref[0, 0:_LO, :] = xb[0:_LO, :]
    o_ref[0, _LO:_HI, :] = jnp.where(interior, blended, center)
    o_ref[0, _HI:, :] = xb[_HI:, :]

    imin = jnp.min(jnp.where(interior, cell, 1.0))

    @pl.when(b == 0)
    def _init():
        m_ref[...] = cell
        am_ref[0, 0] = imin

    @pl.when(b != 0)
    def _acc():
        m_ref[...] = jnp.maximum(m_ref[...], cell)
        am_ref[0, 0] = jnp.maximum(am_ref[0, 0], imin)


def _fixup_kernel(m_ref, am_ref, x_ref, oin_ref, o_ref, sem):
    del oin_ref

    @pl.when(am_ref[0, 0] < 0.5)
    def _scan():
        def body(c, carry):
            i = c // (_GW - 2) + 1
            j = c % (_GW - 2) + 1
            p = i * _GW + j

            @pl.when(m_ref[p - _LO, 0] < 0.5)
            def _restore():
                cp = pltpu.make_async_copy(
                    x_ref.at[:, pl.ds(p, 1), :],
                    o_ref.at[:, pl.ds(p, 1), :],
                    sem,
                )
                cp.start()
                cp.wait()

            return carry

        jax.lax.fori_loop(0, _NI, body, 0)


def kernel(x):
    out1, mask, allmin = pl.pallas_call(
        _fused_kernel,
        grid=(_B,),
        in_specs=[pl.BlockSpec((1, _N, _D), lambda b: (b, 0, 0))],
        out_specs=[
            pl.BlockSpec((1, _N, _D), lambda b: (b, 0, 0)),
            pl.BlockSpec((_M, 1), lambda b: (0, 0)),
            pl.BlockSpec(memory_space=pltpu.SMEM),
        ],
        out_shape=[
            jax.ShapeDtypeStruct((_B, _N, _D), jnp.float32),
            jax.ShapeDtypeStruct((_M, 1), jnp.float32),
            jax.ShapeDtypeStruct((1, 1), jnp.float32),
        ],
    )(x)
    out = pl.pallas_call(
        _fixup_kernel,
        in_specs=[
            pl.BlockSpec(memory_space=pltpu.SMEM),
            pl.BlockSpec(memory_space=pltpu.SMEM),
            pl.BlockSpec(memory_space=pl.ANY),
            pl.BlockSpec(memory_space=pl.ANY),
        ],
        out_specs=pl.BlockSpec(memory_space=pl.ANY),
        out_shape=jax.ShapeDtypeStruct((_B, _N, _D), jnp.float32),
        scratch_shapes=[pltpu.SemaphoreType.DMA],
        input_output_aliases={3: 0},
    )(mask, allmin, x, out1)
    return out


# speedup vs baseline: 11.6091x; 1.3437x over previous
import jax
import jax.numpy as jnp
from jax.experimental import pallas as pl
from jax.experimental.pallas import tpu as pltpu

_B, _GH, _GW, _D = 64, 24, 24, 768
_N = _GH * _GW
_LO, _HI = _GW, _N - _GW
_M = _HI - _LO
_NI = (_GH - 2) * (_GW - 2)
_THRESH_SQ = 1.0
_S = 0.5
_BB = 4


def _fused_kernel(x_ref, o_ref, m_ref, am_ref):
    g = pl.program_id(0)
    cell_acc = None
    for k in range(_BB):
        xb = x_ref[k]
        d1 = xb[1:, :] - xb[:-1, :]
        s1 = jnp.sum(d1 * d1, axis=1, keepdims=True)
        d24 = xb[_GW:, :] - xb[:-_GW, :]
        s24 = jnp.sum(d24 * d24, axis=1, keepdims=True)
        e1 = (s1 > _THRESH_SQ).astype(jnp.float32)
        e24 = (s24 > _THRESH_SQ).astype(jnp.float32)
        cell = jnp.maximum(
            jnp.maximum(e1[_LO - 1:_HI - 1, :], e1[_LO:_HI, :]),
            jnp.maximum(e24[0:_M, :], e24[_LO:_HI, :]),
        )
        cell_acc = cell if cell_acc is None else jnp.maximum(cell_acc, cell)

        center = xb[_LO:_HI, :]
        avg = (xb[_LO - _GW:_HI - _GW, :] + xb[_LO + _GW:_HI + _GW, :]
               + xb[_LO - 1:_HI - 1, :] + xb[_LO + 1:_HI + 1, :]) * 0.25
        o_ref[k, 0:_LO, :] = xb[0:_LO, :]
        o_ref[k, _LO:_HI, :] = (1.0 - _S) * center + _S * avg
        o_ref[k, _HI:, :] = xb[_HI:, :]
        for i in range(1, _GH - 1):
            p0 = i * _GW
            p1 = i * _GW + _GW - 1
            o_ref[k, p0:p0 + 1, :] = xb[p0:p0 + 1, :]
            o_ref[k, p1:p1 + 1, :] = xb[p1:p1 + 1, :]

    p = jax.lax.broadcasted_iota(jnp.int32, (_M, 1), 0) + _LO
    j = p % _GW
    interior = (j >= 1) & (j <= _GW - 2)
    imin = jnp.min(jnp.where(interior, cell_acc, 1.0))

    @pl.when(g == 0)
    def _init():
        m_ref[...] = cell_acc
        am_ref[0, 0] = imin

    @pl.when(g != 0)
    def _acc():
        m_ref[...] = jnp.maximum(m_ref[...], cell_acc)
        am_ref[0, 0] = jnp.maximum(am_ref[0, 0], imin)


def _fixup_kernel(m_ref, am_ref, x_ref, oin_ref, o_ref, sem):
    del oin_ref

    @pl.when(am_ref[0, 0] < 0.5)
    def _scan():
        def body(c, carry):
            i = c // (_GW - 2) + 1
            j = c % (_GW - 2) + 1
            p = i * _GW + j

            @pl.when(m_ref[p - _LO, 0] < 0.5)
            def _restore():
                cp = pltpu.make_async_copy(
                    x_ref.at[:, pl.ds(p, 1), :],
                    o_ref.at[:, pl.ds(p, 1), :],
                    sem,
                )
                cp.start()
                cp.wait()

            return carry

        jax.lax.fori_loop(0, _NI, body, 0)


def kernel(x):
    out1, mask, allmin = pl.pallas_call(
        _fused_kernel,
        grid=(_B // _BB,),
        in_specs=[pl.BlockSpec((_BB, _N, _D), lambda g: (g, 0, 0))],
        out_specs=[
            pl.BlockSpec((_BB, _N, _D), lambda g: (g, 0, 0)),
            pl.BlockSpec((_M, 1), lambda g: (0, 0)),
            pl.BlockSpec(memory_space=pltpu.SMEM),
        ],
        out_shape=[
            jax.ShapeDtypeStruct((_B, _N, _D), jnp.float32),
            jax.ShapeDtypeStruct((_M, 1), jnp.float32),
            jax.ShapeDtypeStruct((1, 1), jnp.float32),
        ],
    )(x)
    out = pl.pallas_call(
        _fixup_kernel,
        in_specs=[
            pl.BlockSpec(memory_space=pltpu.SMEM),
            pl.BlockSpec(memory_space=pltpu.SMEM),
            pl.BlockSpec(memory_space=pl.ANY),
            pl.BlockSpec(memory_space=pl.ANY),
        ],
        out_specs=pl.BlockSpec(memory_space=pl.ANY),
        out_shape=jax.ShapeDtypeStruct((_B, _N, _D), jnp.float32),
        scratch_shapes=[pltpu.SemaphoreType.DMA],
        input_output_aliases={3: 0},
    )(mask, allmin, x, out1)
    return out


# speedup vs baseline: 11.8093x; 1.0172x over previous
import jax
import jax.numpy as jnp
from jax.experimental import pallas as pl
from jax.experimental.pallas import tpu as pltpu

_B, _GH, _GW, _D = 64, 24, 24, 768
_N = _GH * _GW
_LO, _HI = _GW, _N - _GW
_M = _HI - _LO
_NI = (_GH - 2) * (_GW - 2)
_THRESH_SQ = 1.0
_S = 0.5
_BB = 4


def _fused_kernel(x_ref, o_ref, m_ref, am_ref):
    g = pl.program_id(0)
    ones = jnp.ones((_D, 1), jnp.float32)
    cell_acc = None
    for k in range(_BB):
        xb = x_ref[k]
        xs1 = xb[1:, :]
        d1 = xs1 - xb[:-1, :]
        d24 = xb[_GW:, :] - xb[:-_GW, :]
        s1 = jax.lax.dot_general(
            d1 * d1, ones, (((1,), (0,)), ((), ())),
            preferred_element_type=jnp.float32)
        s24 = jax.lax.dot_general(
            d24 * d24, ones, (((1,), (0,)), ((), ())),
            preferred_element_type=jnp.float32)
        smax = jnp.maximum(
            jnp.maximum(s1[_LO - 1:_HI - 1, :], s1[_LO:_HI, :]),
            jnp.maximum(s24[0:_M, :], s24[_LO:_HI, :]),
        )
        cell = (smax > _THRESH_SQ).astype(jnp.float32)
        cell_acc = cell if cell_acc is None else jnp.maximum(cell_acc, cell)

        center = xb[_LO:_HI, :]
        nsum = (xb[_LO - _GW:_HI - _GW, :] + xb[_LO + _GW:_HI + _GW, :]
                + xb[_LO - 1:_HI - 1, :] + xs1[_LO:_HI, :])
        o_ref[k, 0:_LO, :] = xb[0:_LO, :]
        o_ref[k, _LO:_HI, :] = (center + 0.25 * nsum) * _S
        o_ref[k, _HI:, :] = xb[_HI:, :]
        for i in range(1, _GH - 1):
            p0 = i * _GW
            p1 = i * _GW + _GW - 1
            o_ref[k, p0:p0 + 1, :] = xb[p0:p0 + 1, :]
            o_ref[k, p1:p1 + 1, :] = xb[p1:p1 + 1, :]

    p = jax.lax.broadcasted_iota(jnp.int32, (_M, 1), 0) + _LO
    j = p % _GW
    interior = (j >= 1) & (j <= _GW - 2)
    imin = jnp.min(jnp.where(interior, cell_acc, 1.0))

    @pl.when(g == 0)
    def _init():
        m_ref[...] = cell_acc
        am_ref[0, 0] = imin

    @pl.when(g != 0)
    def _acc():
        m_ref[...] = jnp.maximum(m_ref[...], cell_acc)
        am_ref[0, 0] = jnp.maximum(am_ref[0, 0], imin)


def _fixup_kernel(m_ref, am_ref, x_ref, oin_ref, o_ref, sem):
    del oin_ref

    @pl.when(am_ref[0, 0] < 0.5)
    def _scan():
        def body(c, carry):
            i = c // (_GW - 2) + 1
            j = c % (_GW - 2) + 1
            p = i * _GW + j

            @pl.when(m_ref[p - _LO, 0] < 0.5)
            def _restore():
                cp = pltpu.make_async_copy(
                    x_ref.at[:, pl.ds(p, 1), :],
                    o_ref.at[:, pl.ds(p, 1), :],
                    sem,
                )
                cp.start()
                cp.wait()

            return carry

        jax.lax.fori_loop(0, _NI, body, 0)


def kernel(x):
    out1, mask, allmin = pl.pallas_call(
        _fused_kernel,
        grid=(_B // _BB,),
        in_specs=[pl.BlockSpec((_BB, _N, _D), lambda g: (g, 0, 0))],
        out_specs=[
            pl.BlockSpec((_BB, _N, _D), lambda g: (g, 0, 0)),
            pl.BlockSpec((_M, 1), lambda g: (0, 0)),
            pl.BlockSpec(memory_space=pltpu.SMEM),
        ],
        out_shape=[
            jax.ShapeDtypeStruct((_B, _N, _D), jnp.float32),
            jax.ShapeDtypeStruct((_M, 1), jnp.float32),
            jax.ShapeDtypeStruct((1, 1), jnp.float32),
        ],
    )(x)
    out = pl.pallas_call(
        _fixup_kernel,
        in_specs=[
            pl.BlockSpec(memory_space=pltpu.SMEM),
            pl.BlockSpec(memory_space=pltpu.SMEM),
            pl.BlockSpec(memory_space=pl.ANY),
            pl.BlockSpec(memory_space=pl.ANY),
        ],
        out_specs=pl.BlockSpec(memory_space=pl.ANY),
        out_shape=jax.ShapeDtypeStruct((_B, _N, _D), jnp.float32),
        scratch_shapes=[pltpu.SemaphoreType.DMA],
        input_output_aliases={3: 0},
    )(mask, allmin, x, out1)
    return out
